# Initial kernel scaffold; baseline (speedup 1.0000x reference)
import jax
import jax.numpy as jnp
from jax.experimental import pallas as pl


def kernel(nodes_abs, nodes_norm, shift_value, seq_list, scenes, pednum, W_in, b_in, W_g, W_out):
    raise NotImplementedError("write your pallas kernel here")


# speedup vs baseline: 8.4421x; 8.4421x over previous
import jax
import jax.numpy as jnp
from jax.experimental import pallas as pl
from jax.experimental.pallas import tpu as pltpu

SEQ_LENGTH = 20
EMB = 32


def _pfa_kernel(xt_ref, seq_ref, w_in_t_ref, b_ref, w_g_t_ref, w_out_t_ref,
                out_ref):
    nb = out_ref.shape[2]
    w0 = w_in_t_ref[:, 0:1]
    w1 = w_in_t_ref[:, 1:2]
    b = b_ref[:, 0:1]
    w_g_t = w_g_t_ref[:, :]
    w_out_t = w_out_t_ref[:, :]
    s = jnp.zeros((EMB, nb), jnp.float32)
    m = jnp.ones((1, nb), jnp.bool_)
    for f in range(SEQ_LENGTH - 1):
        x = xt_ref[f]
        a = jnp.maximum(w0 * x[0:1, :] + w1 * x[1:2, :] + b, 0.0)
        if f == 0:
            h = a
        else:
            sm = s * jnp.float32(1.0 / f)
            h = a + jax.lax.dot_general(
                w_g_t, sm, (((1,), (0,)), ((), ())),
                preferred_element_type=jnp.float32)
        m = jnp.logical_and(m, seq_ref[f:f + 1, :] > 0.0)
        o = jax.lax.dot_general(
            w_out_t, h, (((1,), (0,)), ((), ())),
            preferred_element_type=jnp.float32)
        out_ref[f] = jnp.where(m, o, 0.0)
        s = s + jnp.where(m, h, 0.0)
    out_ref[SEQ_LENGTH - 1] = jnp.zeros((2, nb), jnp.float32)


def kernel(nodes_abs, nodes_norm, shift_value, seq_list, scenes, pednum,
           W_in, b_in, W_g, W_out):
    T, N = nodes_norm.shape[0], nodes_norm.shape[1]
    nb = min(N, 2048)
    grid = N // nb
    xt = jnp.transpose(nodes_norm, (0, 2, 1))
    out_t = pl.pallas_call(
        _pfa_kernel,
        grid=(grid,),
        in_specs=[
            pl.BlockSpec((T, 2, nb), lambda i: (0, 0, i)),
            pl.BlockSpec((T, nb), lambda i: (0, i)),
            pl.BlockSpec((EMB, 2), lambda i: (0, 0)),
            pl.BlockSpec((EMB, 1), lambda i: (0, 0)),
            pl.BlockSpec((EMB, EMB), lambda i: (0, 0)),
            pl.BlockSpec((2, EMB), lambda i: (0, 0)),
        ],
        out_specs=pl.BlockSpec((T, 2, nb), lambda i: (0, 0, i)),
        out_shape=jax.ShapeDtypeStruct((T, 2, N), jnp.float32),
        compiler_params=pltpu.CompilerParams(
            dimension_semantics=("parallel",)),
    )(xt, seq_list, W_in.T, b_in.reshape(EMB, 1), W_g.T, W_out.T)
    return jnp.transpose(out_t, (0, 2, 1))
